# Initial kernel scaffold; baseline (speedup 1.0000x reference)
import functools

import jax
import jax.numpy as jnp
from jax import lax
from jax.experimental import pallas as pl
from jax.experimental.pallas import tpu as pltpu
from jax.experimental.pallas import tpu_sc as plsc

_N = 10000
_E = 320000
_KD = 64
_MD = 16
_TD = 80

_NC = 2
_NS = 16
_NW = _NC * _NS
_CH = 128
_NCH = 79
_EPW = _NCH * _CH
_EP = _NW * _EPW
_NPT = _N // _NS

_BE = 2048
_GE = _EP // _BE
_BN = 1000
_GN = _N // _BN


def _silu(x):
    return x * jax.nn.sigmoid(x)


def _dot(a, b):
    return jnp.dot(a, b, preferred_element_type=jnp.float32)


def _onehot(ids, k):
    io = lax.broadcasted_iota(jnp.int32, (ids.shape[0], k), 1)
    return (ids == io).astype(jnp.float32)


def _pre_body(aid, rid, hid, arid, chg, crd,
              ea, er, eh, ear, cw, cb,
              w1, b1, w2, b2, w3, b3, out_ref):
    emb = jnp.concatenate([
        _dot(_onehot(aid[...], 10), ea[...]),
        _dot(_onehot(rid[...], 2), er[...]),
        _dot(_onehot(hid[...], 4), eh[...]),
        _dot(_onehot(arid[...], 2), ear[...]),
        chg[...] * cw[...] + cb[...],
    ], axis=1)
    h = _silu(_dot(emb, w1[...]) + b1[...])
    h = _silu(_dot(h, w2[...]) + b2[...])
    h = _silu(_dot(h, w3[...]) + b3[...])
    pad = jnp.zeros((h.shape[0], _TD - _KD - 3), jnp.float32)
    out_ref[...] = jnp.concatenate([h, crd[...], pad], axis=1)


def _pre_call(aid, rid, hid, arid, chg, crd, p):
    col = lambda: pl.BlockSpec((_BN, 1), lambda i: (i, 0))
    full = lambda r, c: pl.BlockSpec((r, c), lambda i: (0, 0))
    return pl.pallas_call(
        _pre_body,
        grid=(_GN,),
        in_specs=[
            col(), col(), col(), col(), col(),
            pl.BlockSpec((_BN, 3), lambda i: (i, 0)),
            full(10, 64), full(2, 64), full(4, 64), full(2, 64),
            full(1, 64), full(1, 64),
            full(320, 128), full(1, 128),
            full(128, 64), full(1, 64),
            full(64, 64), full(1, 64),
        ],
        out_specs=pl.BlockSpec((_BN, _TD), lambda i: (i, 0)),
        out_shape=jax.ShapeDtypeStruct((_N, _TD), jnp.float32),
    )(aid, rid, hid, arid, chg, crd,
      p["atom_em"], p["ring_em"], p["hybr_em"], p["arom_em"],
      p["chrg"]["W"], p["chrg"]["b"].reshape(1, -1),
      p["pre1"]["W"], p["pre1"]["b"].reshape(1, -1),
      p["pre2"]["W"], p["pre2"]["b"].reshape(1, -1),
      p["pre3"]["W"], p["pre3"]["b"].reshape(1, -1))


def _gather_body(t_hbm, dst_hbm, src_hbm, gd_hbm, gs_hbm,
                 idx_d, idx_s, row_d, row_s, sem_d, sem_s):
    w = lax.axis_index("s") * _NC + lax.axis_index("c")
    base = w * _EPW

    def body(i, carry):
        off = base + i * _CH
        pltpu.sync_copy(dst_hbm.at[pl.ds(off, _CH)], idx_d)
        pltpu.sync_copy(src_hbm.at[pl.ds(off, _CH)], idx_s)
        cd = pltpu.async_copy(t_hbm.at[idx_d], row_d, sem_d)
        cs = pltpu.async_copy(t_hbm.at[idx_s], row_s, sem_s)
        cd.wait()
        cs.wait()
        pltpu.sync_copy(row_d, gd_hbm.at[pl.ds(off, _CH)])
        pltpu.sync_copy(row_s, gs_hbm.at[pl.ds(off, _CH)])
        return carry

    lax.fori_loop(0, _NCH, body, 0)


_gather_call = functools.partial(
    pl.kernel,
    _gather_body,
    out_type=(jax.ShapeDtypeStruct((_EP, _TD), jnp.float32),
              jax.ShapeDtypeStruct((_EP, _TD), jnp.float32)),
    mesh=plsc.VectorSubcoreMesh(core_axis_name="c", subcore_axis_name="s"),
    scratch_types=[
        pltpu.VMEM((_CH,), jnp.int32),
        pltpu.VMEM((_CH,), jnp.int32),
        pltpu.VMEM((_CH, _TD), jnp.float32),
        pltpu.VMEM((_CH, _TD), jnp.float32),
        pltpu.SemaphoreType.DMA,
        pltpu.SemaphoreType.DMA,
    ],
)()


def _scatter_body(m_hbm, dst_hbm, z_hbm, agg_hbm, idx_v, row_v, acc_sh):
    c = lax.axis_index("c")
    s = lax.axis_index("s")
    pltpu.sync_copy(z_hbm, acc_sh.at[pl.ds(s * _NPT, _NPT)])
    plsc.subcore_barrier()
    base = (c * _NS + s) * _EPW

    def body(i, carry):
        off = base + i * _CH
        pltpu.sync_copy(dst_hbm.at[pl.ds(off, _CH)], idx_v)
        pltpu.sync_copy(m_hbm.at[pl.ds(off, _CH)], row_v)
        pltpu.sync_copy(row_v, acc_sh.at[idx_v], add=True)
        return carry

    lax.fori_loop(0, _NCH, body, 0)
    plsc.subcore_barrier()
    pltpu.sync_copy(acc_sh.at[pl.ds(s * _NPT, _NPT)],
                    agg_hbm.at[pl.ds(c * _N + s * _NPT, _NPT)])


_scatter_call = functools.partial(
    pl.kernel,
    _scatter_body,
    out_type=jax.ShapeDtypeStruct((_NC * _N, _MD), jnp.float32),
    mesh=plsc.VectorSubcoreMesh(core_axis_name="c", subcore_axis_name="s"),
    scratch_types=[
        pltpu.VMEM((_CH,), jnp.int32),
        pltpu.VMEM((_CH, _MD), jnp.float32),
        pltpu.VMEM_SHARED((_N, _MD), jnp.float32),
    ],
)()


def _edge_body(gd, gs, w1d, w1s, w1r, b1, w2, b2, out_ref):
    fd = gd[:, :_KD]
    fs = gs[:, :_KD]
    diff = gd[...] - gs[...]
    lane = lax.broadcasted_iota(jnp.int32, (_BE, _TD), 1)
    sel = (lane >= _KD) & (lane < _KD + 3)
    rd = jnp.sum(jnp.where(sel, diff * diff, 0.0), axis=1, keepdims=True)
    m = _dot(fd, w1d[...]) + _dot(fs, w1s[...]) + rd * w1r[...] + b1[...]
    m = _silu(m)
    m = _silu(_dot(m, w2[...]) + b2[...])
    eid = lax.broadcasted_iota(jnp.int32, (_BE, _MD), 0) + pl.program_id(0) * _BE
    out_ref[...] = jnp.where(eid < _E, m, 0.0)


def _edge_call(gd, gs, kp):
    w1 = kp["e1"]["W"]
    full = lambda r, c: pl.BlockSpec((r, c), lambda i: (0, 0))
    return pl.pallas_call(
        _edge_body,
        grid=(_GE,),
        in_specs=[
            pl.BlockSpec((_BE, _TD), lambda i: (i, 0)),
            pl.BlockSpec((_BE, _TD), lambda i: (i, 0)),
            full(_KD, 258), full(_KD, 258), full(1, 258), full(1, 258),
            full(258, _MD), full(1, _MD),
        ],
        out_specs=pl.BlockSpec((_BE, _MD), lambda i: (i, 0)),
        out_shape=jax.ShapeDtypeStruct((_EP, _MD), jnp.float32),
    )(gd, gs, w1[:_KD], w1[_KD:2 * _KD], w1[2 * _KD:],
      kp["e1"]["b"].reshape(1, -1), kp["e2"]["W"], kp["e2"]["b"].reshape(1, -1))


def _node_body(t, agg, w1, b1, w2, b2, out_ref):
    f = t[:, :_KD]
    ma = agg[0] + agg[1]
    x = jnp.concatenate([f, ma], axis=1)
    h = _silu(_dot(x, w1[...]) + b1[...])
    nh = _dot(h, w2[...]) + b2[...]
    out_ref[...] = jnp.concatenate([f + nh, t[:, _KD:]], axis=1)


def _node_call(t, agg, kp):
    full = lambda r, c: pl.BlockSpec((r, c), lambda i: (0, 0))
    return pl.pallas_call(
        _node_body,
        grid=(_GN,),
        in_specs=[
            pl.BlockSpec((_BN, _TD), lambda i: (i, 0)),
            pl.BlockSpec((_NC, _BN, _MD), lambda i: (0, i, 0)),
            full(_KD + _MD, 128), full(1, 128),
            full(128, _KD), full(1, _KD),
        ],
        out_specs=pl.BlockSpec((_BN, _TD), lambda i: (i, 0)),
        out_shape=jax.ShapeDtypeStruct((_N, _TD), jnp.float32),
    )(t, agg, kp["n1"]["W"], kp["n1"]["b"].reshape(1, -1),
      kp["n2"]["W"], kp["n2"]["b"].reshape(1, -1))


def _post_body(t1, t2, t3, w1, b1, w2, b2, w3, b3, w4, b4, out_ref):
    x = jnp.concatenate([t1[:, :_KD], t2[:, :_KD], t3[:, :_KD]], axis=1)
    h = _silu(_dot(x, w1[...]) + b1[...])
    h = _silu(_dot(h, w2[...]) + b2[...])
    h = _silu(_dot(h, w3[...]) + b3[...])
    logit = jnp.sum(h * w4[...], axis=1, keepdims=True) + b4[...]
    out_ref[...] = jax.nn.sigmoid(logit)


def _post_call(t1, t2, t3, p):
    full = lambda r, c: pl.BlockSpec((r, c), lambda i: (0, 0))
    blk = lambda: pl.BlockSpec((_BN, _TD), lambda i: (i, 0))
    return pl.pallas_call(
        _post_body,
        grid=(_GN,),
        in_specs=[
            blk(), blk(), blk(),
            full(192, 512), full(1, 512),
            full(512, 512), full(1, 512),
            full(512, 512), full(1, 512),
            full(1, 512), full(1, 1),
        ],
        out_specs=pl.BlockSpec((_BN, 1), lambda i: (i, 0)),
        out_shape=jax.ShapeDtypeStruct((_N, 1), jnp.float32),
    )(t1, t2, t3,
      p["post1"]["W"], p["post1"]["b"].reshape(1, -1),
      p["post2"]["W"], p["post2"]["b"].reshape(1, -1),
      p["post3"]["W"], p["post3"]["b"].reshape(1, -1),
      p["post4"]["W"].reshape(1, -1), p["post4"]["b"].reshape(1, 1))


def kernel(atom_id, ring_id, hybr_id, arom_id, charges, crds_3d, edge_index,
           params):
    src = edge_index[0].astype(jnp.int32)
    dst = edge_index[1].astype(jnp.int32)
    padz = jnp.zeros((_EP - _E,), jnp.int32)
    src_p = jnp.concatenate([src, padz])
    dst_p = jnp.concatenate([dst, padz])
    zrows = jnp.zeros((_NPT, _MD), jnp.float32)

    t = _pre_call(atom_id.astype(jnp.int32).reshape(-1, 1),
                  ring_id.astype(jnp.int32).reshape(-1, 1),
                  hybr_id.astype(jnp.int32).reshape(-1, 1),
                  arom_id.astype(jnp.int32).reshape(-1, 1),
                  charges, crds_3d, params)

    feats = []
    for kp in params["kernels"]:
        gd, gs = _gather_call(t, dst_p, src_p)
        m = _edge_call(gd, gs, kp)
        agg = _scatter_call(m, dst_p, zrows).reshape(_NC, _N, _MD)
        t = _node_call(t, agg, kp)
        feats.append(t)

    out = _post_call(feats[0], feats[1], feats[2], params)
    return out[:, 0]


# speedup vs baseline: 2.2119x; 2.2119x over previous
import functools

import jax
import jax.numpy as jnp
from jax import lax
from jax.experimental import pallas as pl
from jax.experimental.pallas import tpu as pltpu
from jax.experimental.pallas import tpu_sc as plsc

_N = 10000
_E = 320000
_KD = 64
_MD = 16
_TD = 128

_NC = 2
_NS = 16
_NW = _NC * _NS
_CH = 128
_NCH = 79
_EPW = _NCH * _CH
_EP = _NW * _EPW
_NA = 10240
_NPT = _NA // _NS

_BE = 2048
_GE = _EP // _BE
_BN = 1000
_GN = _N // _BN


def _silu(x):
    return x * jax.nn.sigmoid(x)


def _dot(a, b):
    return jnp.dot(a, b, preferred_element_type=jnp.float32)


def _onehot(ids, k):
    io = lax.broadcasted_iota(jnp.int32, (ids.shape[0], k), 1)
    return (ids == io).astype(jnp.float32)


def _pre_body(aid, rid, hid, arid, chg, crd,
              ea, er, eh, ear, cw, cb,
              w1, b1, w2, b2, w3, b3, out_ref):
    emb = jnp.concatenate([
        _dot(_onehot(aid[...], 10), ea[...]),
        _dot(_onehot(rid[...], 2), er[...]),
        _dot(_onehot(hid[...], 4), eh[...]),
        _dot(_onehot(arid[...], 2), ear[...]),
        chg[...] * cw[...] + cb[...],
    ], axis=1)
    h = _silu(_dot(emb, w1[...]) + b1[...])
    h = _silu(_dot(h, w2[...]) + b2[...])
    h = _silu(_dot(h, w3[...]) + b3[...])
    pad = jnp.zeros((h.shape[0], _TD - _KD - 3), jnp.float32)
    out_ref[...] = jnp.concatenate([h, crd[...], pad], axis=1)


def _pre_call(aid, rid, hid, arid, chg, crd, p):
    col = lambda: pl.BlockSpec((_BN, 1), lambda i: (i, 0))
    full = lambda r, c: pl.BlockSpec((r, c), lambda i: (0, 0))
    return pl.pallas_call(
        _pre_body,
        grid=(_GN,),
        in_specs=[
            col(), col(), col(), col(), col(),
            pl.BlockSpec((_BN, 3), lambda i: (i, 0)),
            full(10, 64), full(2, 64), full(4, 64), full(2, 64),
            full(1, 64), full(1, 64),
            full(320, 128), full(1, 128),
            full(128, 64), full(1, 64),
            full(64, 64), full(1, 64),
        ],
        out_specs=pl.BlockSpec((_BN, _TD), lambda i: (i, 0)),
        out_shape=jax.ShapeDtypeStruct((_N, _TD), jnp.float32),
    )(aid, rid, hid, arid, chg, crd,
      p["atom_em"], p["ring_em"], p["hybr_em"], p["arom_em"],
      p["chrg"]["W"], p["chrg"]["b"].reshape(1, -1),
      p["pre1"]["W"], p["pre1"]["b"].reshape(1, -1),
      p["pre2"]["W"], p["pre2"]["b"].reshape(1, -1),
      p["pre3"]["W"], p["pre3"]["b"].reshape(1, -1))


def _gather_body(t_hbm, dst_hbm, src_hbm, gd_hbm, gs_hbm,
                 idx_d, idx_s, row_d, row_s, sem_d, sem_s):
    w = lax.axis_index("s") * _NC + lax.axis_index("c")
    base = w * _EPW

    def body(i, carry):
        off = base + i * _CH
        pltpu.sync_copy(dst_hbm.at[pl.ds(off, _CH)], idx_d)
        pltpu.sync_copy(src_hbm.at[pl.ds(off, _CH)], idx_s)
        cd = pltpu.async_copy(t_hbm.at[idx_d], row_d, sem_d)
        cs = pltpu.async_copy(t_hbm.at[idx_s], row_s, sem_s)
        cd.wait()
        cs.wait()
        pltpu.sync_copy(row_d, gd_hbm.at[pl.ds(off, _CH)])
        pltpu.sync_copy(row_s, gs_hbm.at[pl.ds(off, _CH)])
        return carry

    lax.fori_loop(0, _NCH, body, 0)


@functools.lru_cache(maxsize=None)
def _sc_mesh():
    return plsc.VectorSubcoreMesh(core_axis_name="c", subcore_axis_name="s",
                                  num_cores=_NC, num_subcores=_NS)


@functools.lru_cache(maxsize=None)
def _gather_kernel():
    return pl.kernel(
        _gather_body,
        out_type=(jax.ShapeDtypeStruct((_EP, _TD), jnp.float32),
                  jax.ShapeDtypeStruct((_EP, _TD), jnp.float32)),
        mesh=_sc_mesh(),
        scratch_types=[
            pltpu.VMEM((_CH,), jnp.int32),
            pltpu.VMEM((_CH,), jnp.int32),
            pltpu.VMEM((_CH, _TD), jnp.float32),
            pltpu.VMEM((_CH, _TD), jnp.float32),
            pltpu.SemaphoreType.DMA,
            pltpu.SemaphoreType.DMA,
        ],
    )


def _gather_call(t, dst_p, src_p):
    return _gather_kernel()(t, dst_p, src_p)


def _scatter_body(m_hbm, dst_hbm, z_hbm, agg_hbm, idx_v, row_v, acc_sh):
    c = lax.axis_index("c")
    s = lax.axis_index("s")
    pltpu.sync_copy(z_hbm, acc_sh.at[pl.ds(s * _NPT, _NPT)])
    plsc.subcore_barrier()
    base = (c * _NS + s) * _EPW

    def body(i, carry):
        off = base + i * _CH
        pltpu.sync_copy(dst_hbm.at[pl.ds(off, _CH)], idx_v)
        pltpu.sync_copy(m_hbm.at[pl.ds(off, _CH)], row_v)
        pltpu.sync_copy(row_v, acc_sh.at[idx_v], add=True)
        return carry

    lax.fori_loop(0, _NCH, body, 0)
    plsc.subcore_barrier()
    pltpu.sync_copy(acc_sh.at[pl.ds(s * _NPT, _NPT)],
                    agg_hbm.at[pl.ds(c * _NA + s * _NPT, _NPT)])


@functools.lru_cache(maxsize=None)
def _scatter_kernel():
    return pl.kernel(
        _scatter_body,
        out_type=jax.ShapeDtypeStruct((_NC * _NA, _MD), jnp.float32),
        mesh=_sc_mesh(),
        scratch_types=[
            pltpu.VMEM((_CH,), jnp.int32),
            pltpu.VMEM((_CH, _MD), jnp.float32),
            pltpu.VMEM_SHARED((_NA, _MD), jnp.float32),
        ],
    )


def _scatter_call(m, dst_p, zrows):
    return _scatter_kernel()(m, dst_p, zrows)


def _edge_body(gd, gs, w1d, w1s, w1r, b1, w2, b2, out_ref):
    fd = gd[:, :_KD]
    fs = gs[:, :_KD]
    diff = gd[...] - gs[...]
    lane = lax.broadcasted_iota(jnp.int32, (_BE, _TD), 1)
    sel = (lane >= _KD) & (lane < _KD + 3)
    rd = jnp.sum(jnp.where(sel, diff * diff, 0.0), axis=1, keepdims=True)
    m = _dot(fd, w1d[...]) + _dot(fs, w1s[...]) + rd * w1r[...] + b1[...]
    m = _silu(m)
    m = _silu(_dot(m, w2[...]) + b2[...])
    eid = lax.broadcasted_iota(jnp.int32, (_BE, _MD), 0) + pl.program_id(0) * _BE
    out_ref[...] = jnp.where(eid < _E, m, 0.0)


def _edge_call(gd, gs, kp):
    w1 = kp["e1"]["W"]
    full = lambda r, c: pl.BlockSpec((r, c), lambda i: (0, 0))
    return pl.pallas_call(
        _edge_body,
        grid=(_GE,),
        in_specs=[
            pl.BlockSpec((_BE, _TD), lambda i: (i, 0)),
            pl.BlockSpec((_BE, _TD), lambda i: (i, 0)),
            full(_KD, 258), full(_KD, 258), full(1, 258), full(1, 258),
            full(258, _MD), full(1, _MD),
        ],
        out_specs=pl.BlockSpec((_BE, _MD), lambda i: (i, 0)),
        out_shape=jax.ShapeDtypeStruct((_EP, _MD), jnp.float32),
    )(gd, gs, w1[:_KD], w1[_KD:2 * _KD], w1[2 * _KD:],
      kp["e1"]["b"].reshape(1, -1), kp["e2"]["W"], kp["e2"]["b"].reshape(1, -1))


def _node_body(t, agg, w1, b1, w2, b2, out_ref):
    f = t[:, :_KD]
    ma = agg[0] + agg[1]
    x = jnp.concatenate([f, ma], axis=1)
    h = _silu(_dot(x, w1[...]) + b1[...])
    nh = _dot(h, w2[...]) + b2[...]
    out_ref[...] = jnp.concatenate([f + nh, t[:, _KD:]], axis=1)


def _node_call(t, agg, kp):
    full = lambda r, c: pl.BlockSpec((r, c), lambda i: (0, 0))
    return pl.pallas_call(
        _node_body,
        grid=(_GN,),
        in_specs=[
            pl.BlockSpec((_BN, _TD), lambda i: (i, 0)),
            pl.BlockSpec((_NC, _BN, _MD), lambda i: (0, i, 0)),
            full(_KD + _MD, 128), full(1, 128),
            full(128, _KD), full(1, _KD),
        ],
        out_specs=pl.BlockSpec((_BN, _TD), lambda i: (i, 0)),
        out_shape=jax.ShapeDtypeStruct((_N, _TD), jnp.float32),
    )(t, agg, kp["n1"]["W"], kp["n1"]["b"].reshape(1, -1),
      kp["n2"]["W"], kp["n2"]["b"].reshape(1, -1))


def _post_body(t1, t2, t3, w1, b1, w2, b2, w3, b3, w4, b4, out_ref):
    x = jnp.concatenate([t1[:, :_KD], t2[:, :_KD], t3[:, :_KD]], axis=1)
    h = _silu(_dot(x, w1[...]) + b1[...])
    h = _silu(_dot(h, w2[...]) + b2[...])
    h = _silu(_dot(h, w3[...]) + b3[...])
    logit = jnp.sum(h * w4[...], axis=1, keepdims=True) + b4[...]
    out_ref[...] = jax.nn.sigmoid(logit)


def _post_call(t1, t2, t3, p):
    full = lambda r, c: pl.BlockSpec((r, c), lambda i: (0, 0))
    blk = lambda: pl.BlockSpec((_BN, _TD), lambda i: (i, 0))
    return pl.pallas_call(
        _post_body,
        grid=(_GN,),
        in_specs=[
            blk(), blk(), blk(),
            full(192, 512), full(1, 512),
            full(512, 512), full(1, 512),
            full(512, 512), full(1, 512),
            full(1, 512), full(1, 1),
        ],
        out_specs=pl.BlockSpec((_BN, 1), lambda i: (i, 0)),
        out_shape=jax.ShapeDtypeStruct((_N, 1), jnp.float32),
    )(t1, t2, t3,
      p["post1"]["W"], p["post1"]["b"].reshape(1, -1),
      p["post2"]["W"], p["post2"]["b"].reshape(1, -1),
      p["post3"]["W"], p["post3"]["b"].reshape(1, -1),
      p["post4"]["W"].reshape(1, -1), p["post4"]["b"].reshape(1, 1))


def kernel(atom_id, ring_id, hybr_id, arom_id, charges, crds_3d, edge_index,
           params):
    src = edge_index[0].astype(jnp.int32)
    dst = edge_index[1].astype(jnp.int32)
    padz = jnp.zeros((_EP - _E,), jnp.int32)
    src_p = jnp.concatenate([src, padz])
    dst_p = jnp.concatenate([dst, padz])
    zrows = jnp.zeros((_NPT, _MD), jnp.float32)

    t = _pre_call(atom_id.astype(jnp.int32).reshape(-1, 1),
                  ring_id.astype(jnp.int32).reshape(-1, 1),
                  hybr_id.astype(jnp.int32).reshape(-1, 1),
                  arom_id.astype(jnp.int32).reshape(-1, 1),
                  charges, crds_3d, params)

    feats = []
    for kp in params["kernels"]:
        gd, gs = _gather_call(t, dst_p, src_p)
        m = _edge_call(gd, gs, kp)
        agg = _scatter_call(m, dst_p, zrows).reshape(_NC, _NA, _MD)
        t = _node_call(t, agg, kp)
        feats.append(t)

    out = _post_call(feats[0], feats[1], feats[2], params)
    return out[:, 0]
